# baseline (speedup 1.0000x reference)
import jax
import jax.numpy as jnp
import numpy as np
from jax import lax
from jax.experimental import pallas as pl
from jax.experimental.pallas import tpu as pltpu
from jax.experimental.pallas import tpu_sc as plsc

N = 10000
E = 160000
DIN = 256
H = 128
NC = 2
NG = 128

NT = 16
N_PAD = 10240
E_PAD = 163840
CHUNK = 32
CH_PER_TILE = E_PAD // NT // CHUNK
ROWS_PER_TILE = N_PAD // NT

BN = 1024
NBLK = N_PAD // BN

_f32 = jnp.float32
_HIGH = jax.lax.Precision.HIGHEST
_MASK_HI = np.int32(-65536)


def _bfperm():
    perm = []
    for grp in range(H // 32):
        perm += [32 * grp + 2 * t for t in range(16)]
        perm += [32 * grp + 2 * t + 1 for t in range(16)]
    inv = [0] * H
    for pos, e in enumerate(perm):
        inv[e] = pos
    return inv


_BFCOLS = np.asarray(_bfperm(), np.int32)


def _sc_mesh():
    return plsc.VectorSubcoreMesh(core_axis_name="c", subcore_axis_name="s")


def _deg_body(idx_hbm, deg_hbm, acc_sh, idx_v, ones_v, init_v):
    c = lax.axis_index("c")
    s = lax.axis_index("s")

    def fill(i, ref):
        def body(k, _):
            ref[pl.ds(k * 16, 16)] = jnp.ones((16,), _f32)
            return 0
        lax.fori_loop(0, i, body, 0)

    fill(CHUNK // 16, ones_v)
    fill(ROWS_PER_TILE // 16, init_v)

    pltpu.sync_copy(idx_hbm.at[c, s], idx_v)
    pltpu.sync_copy(init_v, acc_sh.at[pl.ds(s * ROWS_PER_TILE, ROWS_PER_TILE)])
    plsc.subcore_barrier()

    def edge_chunk(j, _):
        pltpu.sync_copy(ones_v, acc_sh.at[idx_v.at[j]], add=True)
        return 0

    lax.fori_loop(0, CH_PER_TILE, edge_chunk, 0)
    plsc.subcore_barrier()
    pltpu.sync_copy(acc_sh.at[pl.ds(s * ROWS_PER_TILE, ROWS_PER_TILE)],
                    deg_hbm.at[c, pl.ds(s * ROWS_PER_TILE, ROWS_PER_TILE)])


def _deg(scidx):
    k = pl.kernel(
        _deg_body,
        out_type=jax.ShapeDtypeStruct((2, N_PAD), _f32),
        mesh=_sc_mesh(),
        scratch_types=[
            pltpu.VMEM_SHARED((N_PAD,), _f32),
            pltpu.VMEM((CH_PER_TILE, CHUNK), jnp.int32),
            pltpu.VMEM((CHUNK,), _f32),
            pltpu.VMEM((ROWS_PER_TILE,), _f32),
        ],
    )
    return k(scidx)


_NBUF = 4
_GDEPTH = 2


def _agg_body(g_hbm, gb_hbm, pidx_hbm, out_hbm, acc_sh, pk, sis, dis,
              bbufs, fbufs, gsems, ssems):
    c = lax.axis_index("c")
    s = lax.axis_index("s")

    pltpu.sync_copy(pidx_hbm.at[c, s], pk)
    pltpu.sync_copy(g_hbm.at[pl.ds(c * N_PAD + s * ROWS_PER_TILE, ROWS_PER_TILE)],
                    acc_sh.at[pl.ds(s * ROWS_PER_TILE, ROWS_PER_TILE)])
    plsc.subcore_barrier()

    def unpack(j, sref, dref):
        def body(k, _):
            pv = pk[j, pl.ds(k * 16, 16)]
            sref[pl.ds(k * 16, 16)] = pv & 0x7FFF
            dref[pl.ds(k * 16, 16)] = pv >> 15
            return 0
        lax.fori_loop(0, CHUNK // 16, body, 0)

    def widen(bbuf, fbuf):
        def row(r, _):
            for grp in range(H // 32):
                w = plsc.bitcast(bbuf[r, pl.ds(32 * grp, 32)], jnp.int32)
                fbuf[r, pl.ds(32 * grp, 16)] = plsc.bitcast(w << 16, _f32)
                fbuf[r, pl.ds(32 * grp + 16, 16)] = plsc.bitcast(
                    w & _MASK_HI, _f32)
            return 0
        lax.fori_loop(0, CHUNK, row, 0)

    def position(p, b):
        sl_new = b
        sl_mid = (b + _NBUF - _GDEPTH) % _NBUF

        @pl.when(jnp.logical_and(p >= _NBUF, p < CH_PER_TILE + _NBUF))
        def _wait_sc():
            pltpu.make_async_copy(fbufs[sl_new], acc_sh.at[dis[sl_new]],
                                  ssems[sl_new]).wait()

        @pl.when(p < CH_PER_TILE)
        def _fire_g():
            unpack(p, sis[sl_new], dis[sl_new])
            pltpu.async_copy(gb_hbm.at[sis[sl_new]], bbufs[sl_new],
                             gsems[sl_new])

        @pl.when(jnp.logical_and(p >= _GDEPTH, p < CH_PER_TILE + _GDEPTH))
        def _fire_sc():
            pltpu.make_async_copy(gb_hbm.at[sis[sl_mid]], bbufs[sl_mid],
                                  gsems[sl_mid]).wait()
            widen(bbufs[sl_mid], fbufs[sl_mid])
            pltpu.async_copy(fbufs[sl_mid], acc_sh.at[dis[sl_mid]],
                             ssems[sl_mid], add=True)

    def super_step(t, _):
        for b in range(_NBUF):
            position(t * _NBUF + b, b)
        return 0

    nsteps = (CH_PER_TILE + 2 * _NBUF - 1) // _NBUF + 1
    lax.fori_loop(0, nsteps, super_step, 0)
    plsc.subcore_barrier()
    pltpu.sync_copy(acc_sh.at[pl.ds(s * ROWS_PER_TILE, ROWS_PER_TILE)],
                    out_hbm.at[pl.ds(c * N_PAD + s * ROWS_PER_TILE, ROWS_PER_TILE)])


def _agg(g_cat, gb_cat, pidx):
    def body(g_hbm, gb_hbm, pidx_hbm, out_hbm, acc_sh, pk,
             si0, si1, si2, si3, di0, di1, di2, di3,
             bb0, bb1, bb2, bb3, fb0, fb1, fb2, fb3,
             g0, g1, g2, g3, s0, s1, s2, s3):
        _agg_body(g_hbm, gb_hbm, pidx_hbm, out_hbm, acc_sh, pk,
                  (si0, si1, si2, si3), (di0, di1, di2, di3),
                  (bb0, bb1, bb2, bb3), (fb0, fb1, fb2, fb3),
                  (g0, g1, g2, g3), (s0, s1, s2, s3))

    k = pl.kernel(
        body,
        out_type=jax.ShapeDtypeStruct((2 * N_PAD, H), _f32),
        mesh=_sc_mesh(),
        compiler_params=pltpu.CompilerParams(use_tc_tiling_on_sc=False,
                                             needs_layout_passes=False),
        scratch_types=[
            pltpu.VMEM_SHARED((N_PAD, H), _f32),
            pltpu.VMEM((CH_PER_TILE, CHUNK), jnp.int32),
        ] + [pltpu.VMEM((CHUNK,), jnp.int32)] * (2 * _NBUF)
          + [pltpu.VMEM((CHUNK, H), jnp.bfloat16)] * _NBUF
          + [pltpu.VMEM((CHUNK, H), _f32)] * _NBUF
          + [pltpu.SemaphoreType.DMA] * (2 * _NBUF),
    )
    return k(g_cat, gb_cat, pidx)


def _front_body(x_ref, w_ref, wp_ref, deg_ref, g_ref, gb_ref, dinv_ref):
    x = x_ref[...]
    dinv = lax.rsqrt(deg_ref[...])
    g_ref[...] = jnp.dot(x, w_ref[0], preferred_element_type=_f32,
                         precision=_HIGH) * dinv
    gb_ref[...] = (jnp.dot(x, wp_ref[0], preferred_element_type=_f32,
                           precision=_HIGH) * dinv).astype(jnp.bfloat16)
    dinv_ref[...] = dinv


def _front(x_p, w1_both, w1p_both, deg_cat):
    return pl.pallas_call(
        _front_body,
        grid=(2 * NBLK,),
        in_specs=[
            pl.BlockSpec((BN, DIN), lambda b: (b % NBLK, 0)),
            pl.BlockSpec((1, DIN, H), lambda b: (b // NBLK, 0, 0)),
            pl.BlockSpec((1, DIN, H), lambda b: (b // NBLK, 0, 0)),
            pl.BlockSpec((BN, 1), lambda b: (b, 0)),
        ],
        out_specs=[
            pl.BlockSpec((BN, H), lambda b: (b, 0)),
            pl.BlockSpec((BN, H), lambda b: (b, 0)),
            pl.BlockSpec((BN, 1), lambda b: (b, 0)),
        ],
        out_shape=[
            jax.ShapeDtypeStruct((2 * N_PAD, H), _f32),
            jax.ShapeDtypeStruct((2 * N_PAD, H), jnp.bfloat16),
            jax.ShapeDtypeStruct((2 * N_PAD, 1), _f32),
        ],
    )(x_p, w1_both, w1p_both, deg_cat)


def _mid_body(a_ref, dinv_ref, w_ref, wp_ref, b_ref, g_ref, gb_ref):
    dinv = dinv_ref[...]
    y = jnp.maximum(a_ref[...] * dinv + b_ref[0], 0.0)
    g_ref[...] = jnp.dot(y, w_ref[0], preferred_element_type=_f32,
                         precision=_HIGH) * dinv
    gb_ref[...] = (jnp.dot(y, wp_ref[0], preferred_element_type=_f32,
                           precision=_HIGH) * dinv).astype(jnp.bfloat16)


def _mid(a_cat, dinv_cat, w2_both, w2p_both, b1_both):
    return pl.pallas_call(
        _mid_body,
        grid=(2 * NBLK,),
        in_specs=[
            pl.BlockSpec((BN, H), lambda b: (b, 0)),
            pl.BlockSpec((BN, 1), lambda b: (b, 0)),
            pl.BlockSpec((1, H, H), lambda b: (b // NBLK, 0, 0)),
            pl.BlockSpec((1, H, H), lambda b: (b // NBLK, 0, 0)),
            pl.BlockSpec((1, 1, H), lambda b: (b // NBLK, 0, 0)),
        ],
        out_specs=[
            pl.BlockSpec((BN, H), lambda b: (b, 0)),
            pl.BlockSpec((BN, H), lambda b: (b, 0)),
        ],
        out_shape=[
            jax.ShapeDtypeStruct((2 * N_PAD, H), _f32),
            jax.ShapeDtypeStruct((2 * N_PAD, H), jnp.bfloat16),
        ],
    )(a_cat, dinv_cat, w2_both, w2p_both, b1_both)


def _final_body(a_td, a_bu, dv_td, dv_bu, b2_ref, bat_ref,
                wc1_ref, bc1_ref, wc2_ref, bc2_ref, out_ref,
                p_td, p_bu, cnt):
    b = pl.program_id(0)

    @pl.when(b == 0)
    def _init():
        p_td[...] = jnp.zeros_like(p_td)
        p_bu[...] = jnp.zeros_like(p_bu)
        cnt[...] = jnp.zeros_like(cnt)

    y_td = jnp.maximum(a_td[...] * dv_td[...] + b2_ref[0], 0.0)
    y_bu = jnp.maximum(a_bu[...] * dv_bu[...] + b2_ref[1], 0.0)
    mt = (bat_ref[...] == lax.broadcasted_iota(jnp.int32, (NG, 1), 0)
          ).astype(_f32)
    p_td[...] += jnp.dot(mt, y_td, preferred_element_type=_f32, precision=_HIGH)
    p_bu[...] += jnp.dot(mt, y_bu, preferred_element_type=_f32, precision=_HIGH)
    cnt[...] += jnp.sum(mt, axis=1, keepdims=True)

    @pl.when(b == NBLK - 1)
    def _head():
        rec = 1.0 / jnp.maximum(cnt[...], 1.0)
        comb = jnp.concatenate([p_td[...] * rec, p_bu[...] * rec], axis=1)
        hc = jnp.maximum(
            jnp.dot(comb, wc1_ref[...], preferred_element_type=_f32,
                    precision=_HIGH) + bc1_ref[...], 0.0)
        out_ref[...] = (jnp.dot(hc, wc2_ref[...], preferred_element_type=_f32,
                                precision=_HIGH) + bc2_ref[...])


def _final(a_cat, dinv_cat, b2_both, batch_row, wc1, bc1, wc2, bc2):
    return pl.pallas_call(
        _final_body,
        grid=(NBLK,),
        in_specs=[
            pl.BlockSpec((BN, H), lambda b: (b, 0)),
            pl.BlockSpec((BN, H), lambda b: (b + NBLK, 0)),
            pl.BlockSpec((BN, 1), lambda b: (b, 0)),
            pl.BlockSpec((BN, 1), lambda b: (b + NBLK, 0)),
            pl.BlockSpec((2, 1, H), lambda b: (0, 0, 0)),
            pl.BlockSpec((1, BN), lambda b: (0, b)),
            pl.BlockSpec((2 * H, H), lambda b: (0, 0)),
            pl.BlockSpec((1, H), lambda b: (0, 0)),
            pl.BlockSpec((H, NC), lambda b: (0, 0)),
            pl.BlockSpec((1, NC), lambda b: (0, 0)),
        ],
        out_specs=pl.BlockSpec((NG, NC), lambda b: (0, 0)),
        out_shape=jax.ShapeDtypeStruct((NG, NC), _f32),
        scratch_shapes=[
            pltpu.VMEM((NG, H), _f32),
            pltpu.VMEM((NG, H), _f32),
            pltpu.VMEM((NG, 1), _f32),
        ],
    )(a_cat, a_cat, dinv_cat, dinv_cat, b2_both, batch_row, wc1, bc1, wc2, bc2)


def kernel(x, edge_index, batch, W1_td, b1_td, W2_td, b2_td,
           W1_bu, b1_bu, W2_bu, b2_bu, Wc1, bc1, Wc2, bc2):
    src, dst = edge_index[0], edge_index[1]
    padv = jnp.full((E_PAD - E,), N, jnp.int32)
    src_p = jnp.concatenate([src, padv])
    dst_p = jnp.concatenate([dst, padv])

    gidx = jnp.stack([src_p, dst_p + N_PAD]).reshape(2, NT, CH_PER_TILE, CHUNK)
    scidx = jnp.stack([dst_p, src_p]).reshape(2, NT, CH_PER_TILE, CHUNK)
    pidx = gidx | (scidx << 15)

    x_p = jnp.pad(x, ((0, N_PAD - N), (0, 0)))
    batch_row = jnp.pad(batch, (0, N_PAD - N),
                        constant_values=NG).reshape(1, N_PAD)

    deg = _deg(scidx)
    deg_cat = deg.reshape(2 * N_PAD, 1)

    w1_both = jnp.stack([W1_td, W1_bu])
    w2_both = jnp.stack([W2_td, W2_bu])
    w1p_both = w1_both[:, :, _BFCOLS]
    w2p_both = w2_both[:, :, _BFCOLS]
    b1_both = jnp.stack([b1_td, b1_bu]).reshape(2, 1, H)
    b2_both = jnp.stack([b2_td, b2_bu]).reshape(2, 1, H)

    g1, gb1, dinv_cat = _front(x_p, w1_both, w1p_both, deg_cat)
    a1 = _agg(g1, gb1, pidx)
    g2, gb2 = _mid(a1, dinv_cat, w2_both, w2p_both, b1_both)
    a2 = _agg(g2, gb2, pidx)
    return _final(a2, dinv_cat, b2_both, batch_row,
                  Wc1, bc1.reshape(1, H), Wc2, bc2.reshape(1, NC))


# speedup vs baseline: 1.0508x; 1.0508x over previous
import jax
import jax.numpy as jnp
import numpy as np
from jax import lax
from jax.experimental import pallas as pl
from jax.experimental.pallas import tpu as pltpu
from jax.experimental.pallas import tpu_sc as plsc

N = 10000
E = 160000
DIN = 256
H = 128
HH = H // 2
NC = 2
NG = 128

NT = 16
N_PAD = 10240
E_PAD = 163840
CHUNK = 128
CH_PER_TILE = E_PAD // NT // CHUNK
ROWS_PER_TILE = N_PAD // NT

BN = 1024
NBLK = N_PAD // BN

_f32 = jnp.float32
_HIGH = jax.lax.Precision.HIGHEST


def _sc_mesh():
    return plsc.VectorSubcoreMesh(core_axis_name="c", subcore_axis_name="s")


def _deg_body(idx_hbm, deg_hbm, acc_sh, idx_v, ones_v, init_v):
    c = lax.axis_index("c")
    s = lax.axis_index("s")

    def fill(i, ref):
        def body(k, _):
            ref[pl.ds(k * 16, 16)] = jnp.ones((16,), _f32)
            return 0
        lax.fori_loop(0, i, body, 0)

    fill(CHUNK // 16, ones_v)
    fill(ROWS_PER_TILE // 16, init_v)

    pltpu.sync_copy(idx_hbm.at[c, s], idx_v)
    pltpu.sync_copy(init_v, acc_sh.at[pl.ds(s * ROWS_PER_TILE, ROWS_PER_TILE)])
    plsc.subcore_barrier()

    def edge_chunk(j, _):
        pltpu.sync_copy(ones_v, acc_sh.at[idx_v.at[j]], add=True)
        return 0

    lax.fori_loop(0, CH_PER_TILE, edge_chunk, 0)
    plsc.subcore_barrier()
    pltpu.sync_copy(acc_sh.at[pl.ds(s * ROWS_PER_TILE, ROWS_PER_TILE)],
                    deg_hbm.at[c, pl.ds(s * ROWS_PER_TILE, ROWS_PER_TILE)])


def _deg(scidx):
    k = pl.kernel(
        _deg_body,
        out_type=jax.ShapeDtypeStruct((2, N_PAD), _f32),
        mesh=_sc_mesh(),
        scratch_types=[
            pltpu.VMEM_SHARED((N_PAD,), _f32),
            pltpu.VMEM((CH_PER_TILE, CHUNK), jnp.int32),
            pltpu.VMEM((CHUNK,), _f32),
            pltpu.VMEM((ROWS_PER_TILE,), _f32),
        ],
    )
    return k(scidx)


_NBUF = 6
_GDEPTH = 3

_MASK_HI = np.int32(-65536)


def _agg_body(g_hbm, gb_hbm, pidx_hbm, out_hbm, acc_sh, pk, sis, dis,
              bbufs, fbufs, gsems, ssems):
    c = lax.axis_index("c")
    s = lax.axis_index("s")

    pltpu.sync_copy(pidx_hbm.at[s], pk)
    pltpu.sync_copy(g_hbm.at[pl.ds(c * N_PAD + s * ROWS_PER_TILE, ROWS_PER_TILE)],
                    acc_sh.at[pl.ds(s * ROWS_PER_TILE, ROWS_PER_TILE)])
    plsc.subcore_barrier()

    goff = c * N_PAD

    def unpack(j, sref, dref):
        def body(k, _):
            pv = pk[j, pl.ds(k * 16, 16)]
            sref[pl.ds(k * 16, 16)] = (pv & 0x7FFF) + goff
            dref[pl.ds(k * 16, 16)] = pv >> 15
            return 0
        lax.fori_loop(0, CHUNK // 16, body, 0)

    def widen(bbuf, fbuf):
        def row(r, _):
            for grp in range(HH // 32):
                w = plsc.bitcast(bbuf[r, pl.ds(32 * grp, 32)], jnp.int32)
                fbuf[r, pl.ds(32 * grp, 16)] = plsc.bitcast(w << 16, _f32)
                fbuf[r, pl.ds(32 * grp + 16, 16)] = plsc.bitcast(
                    w & _MASK_HI, _f32)
            return 0
        lax.fori_loop(0, CHUNK, row, 0)

    def position(p, b):
        sl_new = b
        sl_mid = (b + _NBUF - _GDEPTH) % _NBUF

        @pl.when(jnp.logical_and(p >= _NBUF, p < CH_PER_TILE + _NBUF))
        def _wait_sc():
            pltpu.make_async_copy(fbufs[sl_new], acc_sh.at[dis[sl_new]],
                                  ssems[sl_new]).wait()

        @pl.when(p < CH_PER_TILE)
        def _fire_g():
            unpack(p, sis[sl_new], dis[sl_new])
            pltpu.async_copy(gb_hbm.at[sis[sl_new]], bbufs[sl_new],
                             gsems[sl_new])

        @pl.when(jnp.logical_and(p >= _GDEPTH, p < CH_PER_TILE + _GDEPTH))
        def _fire_sc():
            pltpu.make_async_copy(gb_hbm.at[sis[sl_mid]], bbufs[sl_mid],
                                  gsems[sl_mid]).wait()
            widen(bbufs[sl_mid], fbufs[sl_mid])
            pltpu.async_copy(fbufs[sl_mid], acc_sh.at[dis[sl_mid]],
                             ssems[sl_mid], add=True)

    def super_step(t, _):
        for b in range(_NBUF):
            position(t * _NBUF + b, b)
        return 0

    nsteps = (CH_PER_TILE + 2 * _NBUF - 1) // _NBUF + 1
    lax.fori_loop(0, nsteps, super_step, 0)
    plsc.subcore_barrier()
    pltpu.sync_copy(acc_sh.at[pl.ds(s * ROWS_PER_TILE, ROWS_PER_TILE)],
                    out_hbm.at[pl.ds(c * N_PAD + s * ROWS_PER_TILE, ROWS_PER_TILE)])


def _agg(g_half, gb_half, pidx16):
    def body(g_hbm, gb_hbm, pidx_hbm, out_hbm, acc_sh, pk,
             si0, si1, si2, si3, si4, si5, di0, di1, di2, di3, di4, di5,
             bb0, bb1, bb2, bb3, bb4, bb5, fb0, fb1, fb2, fb3, fb4, fb5,
             g0, g1, g2, g3, g4, g5, s0, s1, s2, s3, s4, s5):
        _agg_body(g_hbm, gb_hbm, pidx_hbm, out_hbm, acc_sh, pk,
                  (si0, si1, si2, si3, si4, si5),
                  (di0, di1, di2, di3, di4, di5),
                  (bb0, bb1, bb2, bb3, bb4, bb5),
                  (fb0, fb1, fb2, fb3, fb4, fb5),
                  (g0, g1, g2, g3, g4, g5), (s0, s1, s2, s3, s4, s5))

    k = pl.kernel(
        body,
        out_type=jax.ShapeDtypeStruct((2 * N_PAD, HH), _f32),
        mesh=_sc_mesh(),
        compiler_params=pltpu.CompilerParams(use_tc_tiling_on_sc=False,
                                             needs_layout_passes=False),
        scratch_types=[
            pltpu.VMEM_SHARED((N_PAD, HH), _f32),
            pltpu.VMEM((CH_PER_TILE, CHUNK), jnp.int32),
        ] + [pltpu.VMEM((CHUNK,), jnp.int32)] * (2 * _NBUF)
          + [pltpu.VMEM((CHUNK, HH), jnp.bfloat16)] * _NBUF
          + [pltpu.VMEM((CHUNK, HH), _f32)] * _NBUF
          + [pltpu.SemaphoreType.DMA] * (2 * _NBUF),
    )
    return k(g_half, gb_half, pidx16)


def _split(g):
    return jnp.stack([g[:, :HH], g[:, HH:]])


def _bfperm():
    perm = []
    for grp in range(2):
        perm += [32 * grp + 2 * t for t in range(16)]
        perm += [32 * grp + 2 * t + 1 for t in range(16)]
    inv = [0] * 64
    for pos, e in enumerate(perm):
        inv[e] = pos
    full = [h * 64 + inv[j] for h in range(2) for j in range(64)]
    return full


_BFCOLS = _bfperm()


def _front_body(x_ref, wtd_ref, wbu_ref, wptd_ref, wpbu_ref,
                degtd_ref, degbu_ref,
                gtd_ref, gbu_ref, gbtd_ref, gbbu_ref, dvtd_ref, dvbu_ref):
    x = x_ref[...]
    dv_td = lax.rsqrt(degtd_ref[...])
    dv_bu = lax.rsqrt(degbu_ref[...])
    gtd_ref[...] = _split(jnp.dot(x, wtd_ref[...], preferred_element_type=_f32,
                                  precision=_HIGH) * dv_td)
    gbu_ref[...] = _split(jnp.dot(x, wbu_ref[...], preferred_element_type=_f32,
                                  precision=_HIGH) * dv_bu)
    gbtd_ref[...] = _split(jnp.dot(x, wptd_ref[...], preferred_element_type=_f32,
                                   precision=_HIGH) * dv_td).astype(jnp.bfloat16)
    gbbu_ref[...] = _split(jnp.dot(x, wpbu_ref[...], preferred_element_type=_f32,
                                   precision=_HIGH) * dv_bu).astype(jnp.bfloat16)
    dvtd_ref[...] = dv_td
    dvbu_ref[...] = dv_bu


def _front(x_p, w1_td, w1_bu, w1p_td, w1p_bu, deg_td, deg_bu):
    return pl.pallas_call(
        _front_body,
        grid=(NBLK,),
        in_specs=[
            pl.BlockSpec((BN, DIN), lambda b: (b, 0)),
            pl.BlockSpec((DIN, H), lambda b: (0, 0)),
            pl.BlockSpec((DIN, H), lambda b: (0, 0)),
            pl.BlockSpec((DIN, H), lambda b: (0, 0)),
            pl.BlockSpec((DIN, H), lambda b: (0, 0)),
            pl.BlockSpec((BN, 1), lambda b: (b, 0)),
            pl.BlockSpec((BN, 1), lambda b: (b, 0)),
        ],
        out_specs=[
            pl.BlockSpec((2, BN, HH), lambda b: (0, b, 0)),
            pl.BlockSpec((2, BN, HH), lambda b: (0, b, 0)),
            pl.BlockSpec((2, BN, HH), lambda b: (0, b, 0)),
            pl.BlockSpec((2, BN, HH), lambda b: (0, b, 0)),
            pl.BlockSpec((BN, 1), lambda b: (b, 0)),
            pl.BlockSpec((BN, 1), lambda b: (b, 0)),
        ],
        out_shape=[
            jax.ShapeDtypeStruct((2, N_PAD, HH), _f32),
            jax.ShapeDtypeStruct((2, N_PAD, HH), _f32),
            jax.ShapeDtypeStruct((2, N_PAD, HH), jnp.bfloat16),
            jax.ShapeDtypeStruct((2, N_PAD, HH), jnp.bfloat16),
            jax.ShapeDtypeStruct((N_PAD, 1), _f32),
            jax.ShapeDtypeStruct((N_PAD, 1), _f32),
        ],
    )(x_p, w1_td, w1_bu, w1p_td, w1p_bu, deg_td, deg_bu)


def _mid_body(a_ref, dinv_ref, w_ref, wp_ref, b_ref, g_ref, gb_ref):
    dinv = dinv_ref[...]
    a = jnp.concatenate([a_ref[0], a_ref[1]], axis=1)
    y = jnp.maximum(a * dinv + b_ref[...], 0.0)
    g_ref[...] = _split(jnp.dot(y, w_ref[...], preferred_element_type=_f32,
                                precision=_HIGH) * dinv)
    gb_ref[...] = _split(jnp.dot(y, wp_ref[...], preferred_element_type=_f32,
                                 precision=_HIGH) * dinv).astype(jnp.bfloat16)


def _mid(a_split, dinv, w2, w2p, b1):
    return pl.pallas_call(
        _mid_body,
        grid=(NBLK,),
        in_specs=[
            pl.BlockSpec((2, BN, HH), lambda b: (0, b, 0)),
            pl.BlockSpec((BN, 1), lambda b: (b, 0)),
            pl.BlockSpec((H, H), lambda b: (0, 0)),
            pl.BlockSpec((H, H), lambda b: (0, 0)),
            pl.BlockSpec((1, H), lambda b: (0, 0)),
        ],
        out_specs=[
            pl.BlockSpec((2, BN, HH), lambda b: (0, b, 0)),
            pl.BlockSpec((2, BN, HH), lambda b: (0, b, 0)),
        ],
        out_shape=[
            jax.ShapeDtypeStruct((2, N_PAD, HH), _f32),
            jax.ShapeDtypeStruct((2, N_PAD, HH), jnp.bfloat16),
        ],
    )(a_split, dinv, w2, w2p, b1)


def _final_body(a_td, a_bu, dv_td, dv_bu, b2td_ref, b2bu_ref, bat_ref,
                wc1_ref, bc1_ref, wc2_ref, bc2_ref, out_ref,
                p_td, p_bu, cnt):
    b = pl.program_id(0)

    @pl.when(b == 0)
    def _init():
        p_td[...] = jnp.zeros_like(p_td)
        p_bu[...] = jnp.zeros_like(p_bu)
        cnt[...] = jnp.zeros_like(cnt)

    atd = jnp.concatenate([a_td[0], a_td[1]], axis=1)
    abu = jnp.concatenate([a_bu[0], a_bu[1]], axis=1)
    y_td = jnp.maximum(atd * dv_td[...] + b2td_ref[...], 0.0)
    y_bu = jnp.maximum(abu * dv_bu[...] + b2bu_ref[...], 0.0)
    mt = (bat_ref[...] == lax.broadcasted_iota(jnp.int32, (NG, 1), 0)
          ).astype(_f32)
    p_td[...] += jnp.dot(mt, y_td, preferred_element_type=_f32, precision=_HIGH)
    p_bu[...] += jnp.dot(mt, y_bu, preferred_element_type=_f32, precision=_HIGH)
    cnt[...] += jnp.sum(mt, axis=1, keepdims=True)

    @pl.when(b == NBLK - 1)
    def _head():
        rec = 1.0 / jnp.maximum(cnt[...], 1.0)
        comb = jnp.concatenate([p_td[...] * rec, p_bu[...] * rec], axis=1)
        hc = jnp.maximum(
            jnp.dot(comb, wc1_ref[...], preferred_element_type=_f32,
                    precision=_HIGH) + bc1_ref[...], 0.0)
        out_ref[...] = (jnp.dot(hc, wc2_ref[...], preferred_element_type=_f32,
                                precision=_HIGH) + bc2_ref[...])


def _final(a2_td, a2_bu, dinv_td, dinv_bu, b2_td, b2_bu,
           batch_row, wc1, bc1, wc2, bc2):
    return pl.pallas_call(
        _final_body,
        grid=(NBLK,),
        in_specs=[
            pl.BlockSpec((2, BN, HH), lambda b: (0, b, 0)),
            pl.BlockSpec((2, BN, HH), lambda b: (0, b, 0)),
            pl.BlockSpec((BN, 1), lambda b: (b, 0)),
            pl.BlockSpec((BN, 1), lambda b: (b, 0)),
            pl.BlockSpec((1, H), lambda b: (0, 0)),
            pl.BlockSpec((1, H), lambda b: (0, 0)),
            pl.BlockSpec((1, BN), lambda b: (0, b)),
            pl.BlockSpec((2 * H, H), lambda b: (0, 0)),
            pl.BlockSpec((1, H), lambda b: (0, 0)),
            pl.BlockSpec((H, NC), lambda b: (0, 0)),
            pl.BlockSpec((1, NC), lambda b: (0, 0)),
        ],
        out_specs=pl.BlockSpec((NG, NC), lambda b: (0, 0)),
        out_shape=jax.ShapeDtypeStruct((NG, NC), _f32),
        scratch_shapes=[
            pltpu.VMEM((NG, H), _f32),
            pltpu.VMEM((NG, H), _f32),
            pltpu.VMEM((NG, 1), _f32),
        ],
    )(a2_td, a2_bu, dinv_td, dinv_bu, b2_td, b2_bu,
      batch_row, wc1, bc1, wc2, bc2)


def kernel(x, edge_index, batch, W1_td, b1_td, W2_td, b2_td,
           W1_bu, b1_bu, W2_bu, b2_bu, Wc1, bc1, Wc2, bc2):
    src, dst = edge_index[0], edge_index[1]
    padv = jnp.full((E_PAD - E,), N, jnp.int32)
    src_p = jnp.concatenate([src, padv])
    dst_p = jnp.concatenate([dst, padv])

    pidx_td = (src_p | (dst_p << 15)).reshape(NT, CH_PER_TILE, CHUNK)
    pidx_bu = (dst_p | (src_p << 15)).reshape(NT, CH_PER_TILE, CHUNK)
    scidx = jnp.stack([dst_p, src_p]).reshape(2, NT, CH_PER_TILE, CHUNK)

    x_p = jnp.pad(x, ((0, N_PAD - N), (0, 0)))
    batch_row = jnp.pad(batch, (0, N_PAD - N),
                        constant_values=NG).reshape(1, N_PAD)

    deg = _deg(scidx)
    deg_td = deg[0].reshape(N_PAD, 1)
    deg_bu = deg[1].reshape(N_PAD, 1)

    bfcols = np.asarray(_BFCOLS, np.int32)
    w1p_td = W1_td[:, bfcols]
    w1p_bu = W1_bu[:, bfcols]
    w2p_td = W2_td[:, bfcols]
    w2p_bu = W2_bu[:, bfcols]

    g1_td, g1_bu, gb1_td, gb1_bu, dinv_td, dinv_bu = _front(
        x_p, W1_td, W1_bu, w1p_td, w1p_bu, deg_td, deg_bu)

    a1_td = _agg(g1_td.reshape(2 * N_PAD, HH),
                 gb1_td.reshape(2 * N_PAD, HH), pidx_td).reshape(2, N_PAD, HH)
    a1_bu = _agg(g1_bu.reshape(2 * N_PAD, HH),
                 gb1_bu.reshape(2 * N_PAD, HH), pidx_bu).reshape(2, N_PAD, HH)

    g2_td, gb2_td = _mid(a1_td, dinv_td, W2_td, w2p_td, b1_td.reshape(1, H))
    g2_bu, gb2_bu = _mid(a1_bu, dinv_bu, W2_bu, w2p_bu, b1_bu.reshape(1, H))

    a2_td = _agg(g2_td.reshape(2 * N_PAD, HH),
                 gb2_td.reshape(2 * N_PAD, HH), pidx_td).reshape(2, N_PAD, HH)
    a2_bu = _agg(g2_bu.reshape(2 * N_PAD, HH),
                 gb2_bu.reshape(2 * N_PAD, HH), pidx_bu).reshape(2, N_PAD, HH)

    return _final(a2_td, a2_bu, dinv_td, dinv_bu,
                  b2_td.reshape(1, H), b2_bu.reshape(1, H),
                  batch_row, Wc1, bc1.reshape(1, H), Wc2, bc2.reshape(1, NC))
